# speedup vs baseline: 1.7100x; 1.7100x over previous
import jax
import jax.numpy as jnp
from jax import lax
from jax.experimental import pallas as pl
from jax.experimental.pallas import tpu as pltpu
from jax.experimental.pallas import tpu_sc as plsc

DICT_SIZE = 5120
EMBED_DIM = 512
B_TOTAL = 1024 * 50

NC = 2
NS = 16
NW = NC * NS

B_PER_W = B_TOTAL // NW
CHUNK = 80
N_CHUNKS = B_PER_W // CHUNK


def _embed_body(x_hbm, table_hbm, out_hbm, idx_v, rows_v, sem):
    wid = lax.axis_index("s") * NC + lax.axis_index("c")
    base = wid * B_PER_W
    pltpu.sync_copy(x_hbm.at[pl.ds(base, B_PER_W)], idx_v)
    for j in range(N_CHUNKS):
        idx_slice = idx_v.at[pl.ds(j * CHUNK, CHUNK)]
        pltpu.async_copy(table_hbm.at[idx_slice], rows_v, sem).wait()
        pltpu.sync_copy(rows_v, out_hbm.at[pl.ds(base + j * CHUNK, CHUNK)])


@jax.jit
def _embed(x_flat, table):
    mesh = plsc.VectorSubcoreMesh(core_axis_name="c", subcore_axis_name="s")
    k = pl.kernel(
        _embed_body,
        out_type=jax.ShapeDtypeStruct((B_TOTAL, EMBED_DIM), jnp.float32),
        mesh=mesh,
        scratch_types=[
            pltpu.VMEM((B_PER_W,), jnp.int32),
            pltpu.VMEM((CHUNK, EMBED_DIM), jnp.float32),
            pltpu.SemaphoreType.DMA,
        ],
    )
    return k(x_flat, table)


def kernel(x, table):
    x_flat = x.reshape(-1).astype(jnp.int32)
    out = _embed(x_flat, table)
    return out.reshape(x.shape[0], x.shape[1], EMBED_DIM)


# speedup vs baseline: 1.7962x; 1.0504x over previous
import jax
import jax.numpy as jnp
from jax import lax
from jax.experimental import pallas as pl
from jax.experimental.pallas import tpu as pltpu
from jax.experimental.pallas import tpu_sc as plsc

DICT_SIZE = 5120
EMBED_DIM = 512
B_TOTAL = 1024 * 50

NC = 2
NS = 16
NW = NC * NS

B_PER_W = B_TOTAL // NW
CHUNK = 80
N_CHUNKS = B_PER_W // CHUNK


def _embed_body(x_hbm, table_hbm, out_hbm, idx_v, rows0, rows1, gsem, osem):
    wid = lax.axis_index("s") * NC + lax.axis_index("c")
    base = wid * B_PER_W
    pltpu.sync_copy(x_hbm.at[pl.ds(base, B_PER_W)], idx_v)

    bufs = (rows0, rows1)

    def gather(j):
        idx_slice = idx_v.at[pl.ds(j * CHUNK, CHUNK)]
        return pltpu.async_copy(table_hbm.at[idx_slice], bufs[j % 2], gsem)

    def put(j):
        return pltpu.async_copy(
            bufs[j % 2], out_hbm.at[pl.ds(base + j * CHUNK, CHUNK)], osem
        )

    g = [None] * N_CHUNKS
    o = [None] * N_CHUNKS
    g[0] = gather(0)
    for j in range(N_CHUNKS):
        if j + 1 < N_CHUNKS:
            if j >= 1:
                o[j - 1].wait()
            g[j + 1] = gather(j + 1)
        g[j].wait()
        o[j] = put(j)
    o[N_CHUNKS - 2].wait()
    o[N_CHUNKS - 1].wait()


@jax.jit
def _embed(x_flat, table):
    mesh = plsc.VectorSubcoreMesh(core_axis_name="c", subcore_axis_name="s")
    k = pl.kernel(
        _embed_body,
        out_type=jax.ShapeDtypeStruct((B_TOTAL, EMBED_DIM), jnp.float32),
        mesh=mesh,
        scratch_types=[
            pltpu.VMEM((B_PER_W,), jnp.int32),
            pltpu.VMEM((CHUNK, EMBED_DIM), jnp.float32),
            pltpu.VMEM((CHUNK, EMBED_DIM), jnp.float32),
            pltpu.SemaphoreType.DMA,
            pltpu.SemaphoreType.DMA,
        ],
    )
    return k(x_flat, table)


def kernel(x, table):
    x_flat = x.reshape(-1).astype(jnp.int32)
    out = _embed(x_flat, table)
    return out.reshape(x.shape[0], x.shape[1], EMBED_DIM)
